# baseline (speedup 1.0000x reference)
import functools

import jax
import jax.numpy as jnp
from jax import lax
from jax.experimental import pallas as pl
from jax.experimental.pallas import tpu as pltpu
from jax.experimental.pallas import tpu_sc as plsc

_N = 10000
_E = 320000
_H = 128
_DE = 16
_NG = 64
_NC = 2
_NS = 16
_NW = _NC * _NS
_CH = 128
_Q = 80
_QH = _Q // 2
_EPAD = _NW * _Q * _CH
_NPAD = 10112
_RPT = _NPAD // _NS


def _zero_rows(buf, nrows, ncols, acc, base, total):
    def zbody(i, _):
        buf[i // (ncols // 16), pl.ds((i % (ncols // 16)) * 16, 16)] = (
            jnp.zeros((16,), jnp.float32))
        return 0
    lax.fori_loop(0, nrows * ncols // 16, zbody, 0)
    nfull = total // nrows
    rem = total - nfull * nrows
    for k in range(nfull):
        pltpu.sync_copy(buf, acc.at[pl.ds(base + k * nrows, nrows)])
    if rem:
        pltpu.sync_copy(buf.at[pl.ds(0, rem)],
                        acc.at[pl.ds(base + nfull * nrows, rem)])


_sc_mesh = plsc.VectorSubcoreMesh(core_axis_name="c", subcore_axis_name="s")


@functools.partial(
    pl.kernel,
    out_type=jax.ShapeDtypeStruct((_NC, _NPAD, _H), jnp.float32),
    mesh=_sc_mesh,
    scratch_types=[
        pltpu.VMEM((_QH, _CH), jnp.int32),
        pltpu.VMEM((2, _CH, _H), jnp.float32),
        pltpu.VMEM_SHARED((_NPAD, _H), jnp.float32),
        pltpu.SemaphoreType.DMA,
        pltpu.SemaphoreType.DMA,
    ],
)
def _e_scatter(ee_hbm, dst_hbm, out_hbm, dst_v, ebuf, acc, sem0, sem1):
    c = lax.axis_index("c")
    s = lax.axis_index("s")
    wid = s * _NC + c
    _zero_rows(ebuf.at[0], _CH, _H, acc, s * _RPT, _RPT)
    plsc.subcore_barrier()

    for half in range(2):
        base = half * _QH
        pltpu.sync_copy(dst_hbm.at[wid, pl.ds(base, _QH)], dst_v)
        pltpu.async_copy(ee_hbm.at[wid, base], ebuf.at[0], sem0)

        def body(i, _, base=base):
            j = 2 * i
            pltpu.async_copy(ee_hbm.at[wid, base + j + 1], ebuf.at[1], sem1)
            pltpu.make_async_copy(ee_hbm.at[wid, 0], ebuf.at[0], sem0).wait()
            pltpu.sync_copy(ebuf.at[0], acc.at[dst_v.at[j]], add=True)

            @pl.when(j + 2 < _QH)
            def _():
                pltpu.async_copy(ee_hbm.at[wid, base + j + 2], ebuf.at[0],
                                 sem0)
            pltpu.make_async_copy(ee_hbm.at[wid, 0], ebuf.at[1], sem1).wait()
            pltpu.sync_copy(ebuf.at[1], acc.at[dst_v.at[j + 1]], add=True)
            return 0
        lax.fori_loop(0, _QH // 2, body, 0)
    plsc.subcore_barrier()
    pltpu.sync_copy(acc.at[pl.ds(s * _RPT, _RPT)],
                    out_hbm.at[c, pl.ds(s * _RPT, _RPT)])


@functools.partial(
    pl.kernel,
    out_type=jax.ShapeDtypeStruct((_NC, _NPAD, _H), jnp.float32),
    mesh=_sc_mesh,
    scratch_types=[
        pltpu.VMEM((_QH, _CH), jnp.int32),
        pltpu.VMEM((_QH, _CH), jnp.int32),
        pltpu.VMEM((2, _CH, _H), jnp.float32),
        pltpu.VMEM_SHARED((_NPAD, _H), jnp.float32),
        pltpu.SemaphoreType.DMA,
        pltpu.SemaphoreType.DMA,
    ],
)
def _h_scatter(h_hbm, src_hbm, dst_hbm, out_hbm, src_v, dst_v, rows_v, acc,
               sem0, sem1):
    c = lax.axis_index("c")
    s = lax.axis_index("s")
    wid = s * _NC + c
    _zero_rows(rows_v.at[0], _CH, _H, acc, s * _RPT, _RPT)
    plsc.subcore_barrier()

    for half in range(2):
        base = half * _QH
        pltpu.sync_copy(src_hbm.at[wid, pl.ds(base, _QH)], src_v)
        pltpu.sync_copy(dst_hbm.at[wid, pl.ds(base, _QH)], dst_v)
        pltpu.async_copy(h_hbm.at[src_v.at[0]], rows_v.at[0], sem0)

        def body(i, _):
            j = 2 * i
            pltpu.async_copy(h_hbm.at[src_v.at[j + 1]], rows_v.at[1], sem1)
            pltpu.make_async_copy(h_hbm.at[src_v.at[0]], rows_v.at[0],
                                  sem0).wait()
            pltpu.sync_copy(rows_v.at[0], acc.at[dst_v.at[j]], add=True)

            @pl.when(j + 2 < _QH)
            def _():
                pltpu.async_copy(h_hbm.at[src_v.at[j + 2]], rows_v.at[0],
                                 sem0)
            pltpu.make_async_copy(h_hbm.at[src_v.at[0]], rows_v.at[1],
                                  sem1).wait()
            pltpu.sync_copy(rows_v.at[1], acc.at[dst_v.at[j + 1]], add=True)
            return 0
        lax.fori_loop(0, _QH // 2, body, 0)
    plsc.subcore_barrier()
    pltpu.sync_copy(acc.at[pl.ds(s * _RPT, _RPT)],
                    out_hbm.at[c, pl.ds(s * _RPT, _RPT)])


_EBLK = _EPAD // 16


def _eemb_body(ea_ref, wedge_ref, out_ref):
    out_ref[...] = jnp.dot(ea_ref[...], wedge_ref[...],
                           preferred_element_type=jnp.float32)


_eemb_call = pl.pallas_call(
    _eemb_body,
    grid=(16,),
    in_specs=[pl.BlockSpec((_EBLK, _DE), lambda i: (i, 0)),
              pl.BlockSpec((_DE, _H), lambda i: (0, 0))],
    out_specs=pl.BlockSpec((_EBLK, _H), lambda i: (i, 0)),
    out_shape=jax.ShapeDtypeStruct((_EPAD, _H), jnp.float32),
)


def _prep_body(x0_ref, x1_ref, pe_ref, emb1_ref, emb2_ref, h0_ref, eagg_ref):
    x0 = x0_ref[...]
    x1 = x1_ref[...]
    h0 = jnp.zeros((_N, _H), jnp.float32)
    for k in range(5):
        h0 = h0 + jnp.where(x0 == k, 1.0, 0.0) * emb1_ref[k:k + 1, :]
    for k in range(3):
        h0 = h0 + jnp.where(x1 == k, 1.0, 0.0) * emb2_ref[k:k + 1, :]
    h0_ref[...] = h0
    eagg_ref[...] = pe_ref[0, :_N, :] + pe_ref[1, :_N, :]


_prep_call = pl.pallas_call(
    _prep_body,
    out_shape=[jax.ShapeDtypeStruct((_N, _H), jnp.float32),
               jax.ShapeDtypeStruct((_N, _H), jnp.float32)],
)


def _layer_body(h_ref, p_ref, eagg_ref, w1_ref, b1_ref, w2_ref, b2_ref,
                sc_ref, bi_ref, out_ref, *, last):
    h_in = (h_ref[...] + p_ref[0, :_N, :] + p_ref[1, :_N, :] + eagg_ref[...])
    a = jnp.maximum(
        jnp.dot(h_in, w1_ref[...], preferred_element_type=jnp.float32)
        + b1_ref[...], 0.0)
    z = jnp.dot(a, w2_ref[...], preferred_element_type=jnp.float32) + b2_ref[...]
    mean = jnp.mean(z, axis=0, keepdims=True)
    zc = z - mean
    var = jnp.mean(zc * zc, axis=0, keepdims=True)
    h2 = zc * lax.rsqrt(var + 1e-5) * sc_ref[...] + bi_ref[...]
    out_ref[...] = h2 if last else jnp.maximum(h2, 0.0)


_layer_call = [
    pl.pallas_call(
        functools.partial(_layer_body, last=(l == 2)),
        out_shape=jax.ShapeDtypeStruct((_N, _H), jnp.float32),
    )
    for l in range(3)
]


def _pool_body(h_ref, b_ref, wf_ref, bf_ref, wo_ref, bo_ref, g_ref, l_ref):
    gid = lax.broadcasted_iota(jnp.int32, (_NG, _N), 0)
    m = jnp.where(gid == b_ref[...], 1.0, 0.0)
    cnt = jnp.sum(m, axis=1, keepdims=True)
    gsum = jnp.dot(m, h_ref[...], preferred_element_type=jnp.float32,
                   precision=lax.Precision.HIGHEST)
    g = gsum / jnp.maximum(cnt, 1.0)
    gf = jnp.dot(g, wf_ref[...], preferred_element_type=jnp.float32) + bf_ref[...]
    g_ref[...] = gf
    l_ref[...] = jnp.dot(gf, wo_ref[...],
                         preferred_element_type=jnp.float32) + bo_ref[...]


_pool_call = pl.pallas_call(
    _pool_body,
    out_shape=[jax.ShapeDtypeStruct((_NG, _H), jnp.float32),
               jax.ShapeDtypeStruct((_NG, 2), jnp.float32)],
)


def kernel(x, e_index, e_attr, batch, emb1, emb2, W_edge, W1, b1, W2, b2,
           bn_scale, bn_bias, W_feat, b_feat, W_out, b_out):
    x = x.astype(jnp.int32)
    ei = e_index.astype(jnp.int32)
    batch32 = batch.astype(jnp.int32).reshape(1, _N)
    pad = _EPAD - _E
    srcw = jnp.concatenate(
        [ei[0], jnp.zeros((pad,), jnp.int32)]).reshape(_NW, _Q, _CH)
    dstw = jnp.concatenate(
        [ei[1], jnp.full((pad,), _N, jnp.int32)]).reshape(_NW, _Q, _CH)
    eap = jnp.concatenate([e_attr, jnp.zeros((pad, _DE), jnp.float32)])

    e_emb = _eemb_call(eap, W_edge).reshape(_NW, _Q, _CH, _H)
    pe = _e_scatter(e_emb, dstw)
    h, eagg = _prep_call(x[:, 0:1], x[:, 1:2], pe, emb1, emb2)
    for l in range(3):
        p = _h_scatter(h, srcw, dstw)
        h = _layer_call[l](h, p, eagg, W1[l], b1[l].reshape(1, -1), W2[l],
                           b2[l].reshape(1, -1), bn_scale[l].reshape(1, -1),
                           bn_bias[l].reshape(1, -1))
    g, logits = _pool_call(h, batch32, W_feat, b_feat.reshape(1, -1),
                           W_out, b_out.reshape(1, -1))
    return (g, logits)


# speedup vs baseline: 1.2428x; 1.2428x over previous
import functools

import jax
import jax.numpy as jnp
from jax import lax
from jax.experimental import pallas as pl
from jax.experimental.pallas import tpu as pltpu
from jax.experimental.pallas import tpu_sc as plsc

_N = 10000
_E = 320000
_H = 128
_DE = 16
_NG = 64
_NC = 2
_NS = 16
_NW = _NC * _NS
_CH = 128
_Q = 79
_EPAD = _NW * _Q * _CH
_NPAD = 10112
_RPT = _NPAD // _NS


def _zero_rows(buf, nrows, ncols, acc, base, total):
    def zbody(i, _):
        buf[i // (ncols // 16), pl.ds((i % (ncols // 16)) * 16, 16)] = (
            jnp.zeros((16,), jnp.float32))
        return 0
    lax.fori_loop(0, nrows * ncols // 16, zbody, 0)
    nfull = total // nrows
    rem = total - nfull * nrows
    for k in range(nfull):
        pltpu.sync_copy(buf, acc.at[pl.ds(base + k * nrows, nrows)])
    if rem:
        pltpu.sync_copy(buf.at[pl.ds(0, rem)],
                        acc.at[pl.ds(base + nfull * nrows, rem)])


_sc_mesh = plsc.VectorSubcoreMesh(core_axis_name="c", subcore_axis_name="s")


@functools.partial(
    pl.kernel,
    out_type=jax.ShapeDtypeStruct((_NC, _NPAD, _H), jnp.float32),
    mesh=_sc_mesh,
    scratch_types=[
        pltpu.VMEM((_Q, _CH), jnp.int32),
        pltpu.VMEM((_CH, _H), jnp.float32),
        pltpu.VMEM_SHARED((_NPAD, _H), jnp.float32),
        pltpu.SemaphoreType.DMA,
    ],
)
def _e_scatter(ee_hbm, dst_hbm, out_hbm, dst_v, ebuf, acc, sem):
    c = lax.axis_index("c")
    s = lax.axis_index("s")
    wid = s * _NC + c
    pltpu.sync_copy(dst_hbm.at[wid], dst_v)
    _zero_rows(ebuf, _CH, _H, acc, s * _RPT, _RPT)
    plsc.subcore_barrier()

    def body(j, _):
        pltpu.async_copy(ee_hbm.at[wid, j], ebuf, sem).wait()
        pltpu.sync_copy(ebuf, acc.at[dst_v.at[j]], add=True)
        return 0
    lax.fori_loop(0, _Q, body, 0)
    plsc.subcore_barrier()
    pltpu.sync_copy(acc.at[pl.ds(s * _RPT, _RPT)],
                    out_hbm.at[c, pl.ds(s * _RPT, _RPT)])


@functools.partial(
    pl.kernel,
    out_type=jax.ShapeDtypeStruct((_NC, _NPAD, _H), jnp.float32),
    mesh=_sc_mesh,
    scratch_types=[
        pltpu.VMEM((_Q, _CH), jnp.int32),
        pltpu.VMEM((_Q, _CH), jnp.int32),
        pltpu.VMEM((_CH, _H), jnp.float32),
        pltpu.VMEM_SHARED((_NPAD, _H), jnp.float32),
        pltpu.SemaphoreType.DMA,
    ],
)
def _h_scatter(h_hbm, src_hbm, dst_hbm, out_hbm, src_v, dst_v, rows_v, acc,
               sem):
    c = lax.axis_index("c")
    s = lax.axis_index("s")
    wid = s * _NC + c
    pltpu.sync_copy(src_hbm.at[wid], src_v)
    pltpu.sync_copy(dst_hbm.at[wid], dst_v)
    _zero_rows(rows_v, _CH, _H, acc, s * _RPT, _RPT)
    plsc.subcore_barrier()

    def body(j, _):
        pltpu.async_copy(h_hbm.at[src_v.at[j]], rows_v, sem).wait()
        pltpu.sync_copy(rows_v, acc.at[dst_v.at[j]], add=True)
        return 0
    lax.fori_loop(0, _Q, body, 0)
    plsc.subcore_barrier()
    pltpu.sync_copy(acc.at[pl.ds(s * _RPT, _RPT)],
                    out_hbm.at[c, pl.ds(s * _RPT, _RPT)])


_EBLK = _EPAD // 16


def _eemb_body(ea_ref, wedge_ref, out_ref):
    out_ref[...] = jnp.dot(ea_ref[...], wedge_ref[...],
                           preferred_element_type=jnp.float32)


_eemb_call = pl.pallas_call(
    _eemb_body,
    grid=(16,),
    in_specs=[pl.BlockSpec((_EBLK, _DE), lambda i: (i, 0)),
              pl.BlockSpec((_DE, _H), lambda i: (0, 0))],
    out_specs=pl.BlockSpec((_EBLK, _H), lambda i: (i, 0)),
    out_shape=jax.ShapeDtypeStruct((_EPAD, _H), jnp.float32),
)


def _prep_body(x0_ref, x1_ref, pe_ref, emb1_ref, emb2_ref, h0_ref, eagg_ref):
    x0 = x0_ref[...]
    x1 = x1_ref[...]
    h0 = jnp.zeros((_N, _H), jnp.float32)
    for k in range(5):
        h0 = h0 + jnp.where(x0 == k, 1.0, 0.0) * emb1_ref[k:k + 1, :]
    for k in range(3):
        h0 = h0 + jnp.where(x1 == k, 1.0, 0.0) * emb2_ref[k:k + 1, :]
    h0_ref[...] = h0
    eagg_ref[...] = pe_ref[0, :_N, :] + pe_ref[1, :_N, :]


_prep_call = pl.pallas_call(
    _prep_body,
    out_shape=[jax.ShapeDtypeStruct((_N, _H), jnp.float32),
               jax.ShapeDtypeStruct((_N, _H), jnp.float32)],
)


def _layer_body(h_ref, p_ref, eagg_ref, w1_ref, b1_ref, w2_ref, b2_ref,
                sc_ref, bi_ref, out_ref, *, last):
    h_in = (h_ref[...] + p_ref[0, :_N, :] + p_ref[1, :_N, :] + eagg_ref[...])
    a = jnp.maximum(
        jnp.dot(h_in, w1_ref[...], preferred_element_type=jnp.float32)
        + b1_ref[...], 0.0)
    z = jnp.dot(a, w2_ref[...], preferred_element_type=jnp.float32) + b2_ref[...]
    mean = jnp.mean(z, axis=0, keepdims=True)
    zc = z - mean
    var = jnp.mean(zc * zc, axis=0, keepdims=True)
    h2 = zc * lax.rsqrt(var + 1e-5) * sc_ref[...] + bi_ref[...]
    out_ref[...] = h2 if last else jnp.maximum(h2, 0.0)


_layer_call = [
    pl.pallas_call(
        functools.partial(_layer_body, last=(l == 2)),
        out_shape=jax.ShapeDtypeStruct((_N, _H), jnp.float32),
    )
    for l in range(3)
]


def _pool_body(h_ref, b_ref, wf_ref, bf_ref, wo_ref, bo_ref, g_ref, l_ref):
    gid = lax.broadcasted_iota(jnp.int32, (_NG, _N), 0)
    m = jnp.where(gid == b_ref[...], 1.0, 0.0)
    cnt = jnp.sum(m, axis=1, keepdims=True)
    gsum = jnp.dot(m, h_ref[...], preferred_element_type=jnp.float32,
                   precision=lax.Precision.HIGHEST)
    g = gsum / jnp.maximum(cnt, 1.0)
    gf = jnp.dot(g, wf_ref[...], preferred_element_type=jnp.float32) + bf_ref[...]
    g_ref[...] = gf
    l_ref[...] = jnp.dot(gf, wo_ref[...],
                         preferred_element_type=jnp.float32) + bo_ref[...]


_pool_call = pl.pallas_call(
    _pool_body,
    out_shape=[jax.ShapeDtypeStruct((_NG, _H), jnp.float32),
               jax.ShapeDtypeStruct((_NG, 2), jnp.float32)],
)


def kernel(x, e_index, e_attr, batch, emb1, emb2, W_edge, W1, b1, W2, b2,
           bn_scale, bn_bias, W_feat, b_feat, W_out, b_out):
    x = x.astype(jnp.int32)
    ei = e_index.astype(jnp.int32)
    batch32 = batch.astype(jnp.int32).reshape(1, _N)
    pad = _EPAD - _E
    srcw = jnp.concatenate(
        [ei[0], jnp.zeros((pad,), jnp.int32)]).reshape(_NW, _Q, _CH)
    dstw = jnp.concatenate(
        [ei[1], jnp.full((pad,), _N, jnp.int32)]).reshape(_NW, _Q, _CH)
    eap = jnp.concatenate([e_attr, jnp.zeros((pad, _DE), jnp.float32)])

    e_emb = _eemb_call(eap, W_edge).reshape(_NW, _Q, _CH, _H)
    pe = _e_scatter(e_emb, dstw)
    h, eagg = _prep_call(x[:, 0:1], x[:, 1:2], pe, emb1, emb2)
    for l in range(3):
        p = _h_scatter(h, srcw, dstw)
        h = _layer_call[l](h, p, eagg, W1[l], b1[l].reshape(1, -1), W2[l],
                           b2[l].reshape(1, -1), bn_scale[l].reshape(1, -1),
                           bn_bias[l].reshape(1, -1))
    g, logits = _pool_call(h, batch32, W_feat, b_feat.reshape(1, -1),
                           W_out, b_out.reshape(1, -1))
    return (g, logits)
